# Initial kernel scaffold; baseline (speedup 1.0000x reference)
import functools
import jax
import jax.numpy as jnp
from jax import lax
from jax.experimental import pallas as pl
from jax.experimental.pallas import tpu as pltpu
from jax.experimental.pallas import tpu_sc as plsc

_NU = 25000
_NB = 25000
_E = 800000
_NL = 100000
_D = 64
_NC = 2
_NS = 16
_L = 16

_BINS = 25088
_SL = _BINS // _NS
_CH = 128
_NCH = 391
_EPT = _NCH * _CH
_EPAD = _EPT * _NS

_TPT = 3200
_DCH = _TPT // _CH
_NLP = _TPT * _NS * _NC

_HI = jax.lax.Precision.HIGHEST


def _dot(a, b):
    return jax.lax.dot_general(a, b, (((1,), (0,)), ((), ())),
                               precision=_HI, preferred_element_type=jnp.float32)


def _tc1_body(xu, xb, Wu, bu, Wb, bb, Wlub, blub, Wrub, Wlbu, blbu, Wrbu,
              au, ab, su, sb):
    hu = _dot(xu[...], Wu[...]) + bu[...]
    hb = _dot(xb[...], Wb[...]) + bb[...]
    au[...] = _dot(hu, Wlub[...])
    ab[...] = _dot(hb, Wlbu[...])
    su[...] = _dot(hu, Wrbu[...]) + blbu[...]
    sb[...] = _dot(hb, Wrub[...]) + blub[...]


def _tc1(xu, xb, Wu, bu, Wb, bb, Wlub, blub, Wrub, Wlbu, blbu, Wrbu):
    n = _NU
    bn = 1000
    grid = (n // bn,)
    row = lambda i: (i, 0)
    full = lambda i: (0, 0)
    out = [jax.ShapeDtypeStruct((n, _D), jnp.float32)] * 4
    return pl.pallas_call(
        _tc1_body,
        grid=grid,
        in_specs=[
            pl.BlockSpec((bn, 3), row),
            pl.BlockSpec((bn, 384), row),
            pl.BlockSpec((3, _D), full),
            pl.BlockSpec((1, _D), full),
            pl.BlockSpec((384, _D), full),
            pl.BlockSpec((1, _D), full),
            pl.BlockSpec((_D, _D), full),
            pl.BlockSpec((1, _D), full),
            pl.BlockSpec((_D, _D), full),
            pl.BlockSpec((_D, _D), full),
            pl.BlockSpec((1, _D), full),
            pl.BlockSpec((_D, _D), full),
        ],
        out_specs=[pl.BlockSpec((bn, _D), row)] * 4,
        out_shape=out,
    )(xu, xb, Wu, bu.reshape(1, _D), Wb, bb.reshape(1, _D),
      Wlub, blub.reshape(1, _D), Wrub, Wlbu, blbu.reshape(1, _D), Wrbu)


def _tc2_body(Sb, Su, cb, cu, sb, su, Wlub, blub, Wrub, Wlbu, blbu, Wrbu,
              au2, ab2, sb2, su2):
    hb = jnp.maximum(Sb[...] / jnp.maximum(cb[...], 1.0) + sb[...], 0.0)
    hu = jnp.maximum(Su[...] / jnp.maximum(cu[...], 1.0) + su[...], 0.0)
    au2[...] = _dot(hu, Wlub[...])
    ab2[...] = _dot(hb, Wlbu[...])
    sb2[...] = _dot(hb, Wrub[...]) + blub[...]
    su2[...] = _dot(hu, Wrbu[...]) + blbu[...]


def _tc2(Sb, Su, cb, cu, sb, su, Wlub, blub, Wrub, Wlbu, blbu, Wrbu):
    n = _NU
    bn = 1000
    grid = (n // bn,)
    row = lambda i: (i, 0)
    full = lambda i: (0, 0)
    out = [jax.ShapeDtypeStruct((n, _D), jnp.float32)] * 4
    return pl.pallas_call(
        _tc2_body,
        grid=grid,
        in_specs=[
            pl.BlockSpec((bn, _D), row),
            pl.BlockSpec((bn, _D), row),
            pl.BlockSpec((bn, 1), row),
            pl.BlockSpec((bn, 1), row),
            pl.BlockSpec((bn, _D), row),
            pl.BlockSpec((bn, _D), row),
            pl.BlockSpec((_D, _D), full),
            pl.BlockSpec((1, _D), full),
            pl.BlockSpec((_D, _D), full),
            pl.BlockSpec((_D, _D), full),
            pl.BlockSpec((1, _D), full),
            pl.BlockSpec((_D, _D), full),
        ],
        out_specs=[pl.BlockSpec((bn, _D), row)] * 4,
        out_shape=out,
    )(Sb, Su, cb, cu, sb, su,
      Wlub, blub.reshape(1, _D), Wrub, Wlbu, blbu.reshape(1, _D), Wrbu)


def _tc3_body(Sb, Su, cb, cu, sb, su, dW1, qu, qb):
    zb = Sb[...] / jnp.maximum(cb[...], 1.0) + sb[...]
    zu = Su[...] / jnp.maximum(cu[...], 1.0) + su[...]
    w = dW1[...]
    qu[...] = _dot(zu, w[:_D, :])
    qb[...] = _dot(zb, w[_D:, :])


def _tc3(Sb, Su, cb, cu, sb, su, dW1):
    n = _NU
    bn = 1000
    grid = (n // bn,)
    row = lambda i: (i, 0)
    full = lambda i: (0, 0)
    out = [jax.ShapeDtypeStruct((n, _D), jnp.float32)] * 2
    return pl.pallas_call(
        _tc3_body,
        grid=grid,
        in_specs=[
            pl.BlockSpec((bn, _D), row),
            pl.BlockSpec((bn, _D), row),
            pl.BlockSpec((bn, 1), row),
            pl.BlockSpec((bn, 1), row),
            pl.BlockSpec((bn, _D), row),
            pl.BlockSpec((bn, _D), row),
            pl.BlockSpec((2 * _D, _D), full),
        ],
        out_specs=[pl.BlockSpec((bn, _D), row)] * 2,
        out_shape=out,
    )(Sb, Su, cb, cu, sb, su, dW1)


def _make_segsum(with_counts):
    mesh = plsc.VectorSubcoreMesh(core_axis_name="c", subcore_axis_name="s",
                                  num_cores=_NC, num_subcores=_NS)
    out_type = [jax.ShapeDtypeStruct((_NC, _BINS, _D), jnp.float32)]
    if with_counts:
        out_type.append(jax.ShapeDtypeStruct((_NC, _BINS), jnp.float32))

    @functools.partial(
        pl.kernel,
        out_type=tuple(out_type),
        mesh=mesh,
        scratch_types=[
            pltpu.VMEM_SHARED((_BINS, _D), jnp.float32),
            pltpu.VMEM_SHARED((_BINS,), jnp.float32),
            pltpu.VMEM((_CH,), jnp.int32),
            pltpu.VMEM((_CH,), jnp.int32),
            pltpu.VMEM((_CH, _D), jnp.float32),
            pltpu.VMEM((_CH,), jnp.float32),
            pltpu.SemaphoreType.DMA,
        ],
    )
    def segsum(tab, src, dst, zrow, zcnt, ones_h, *rest):
        if with_counts:
            out, cnt, acc, cacc, idxs, idxd, rows, ones_v, sem = rest
        else:
            out, acc, cacc, idxs, idxd, rows, ones_v, sem = rest
            cnt = None
        c = lax.axis_index("c")
        s = lax.axis_index("s")
        pltpu.sync_copy(zrow, acc.at[pl.ds(s * _SL, _SL), :])
        if with_counts:
            pltpu.sync_copy(zcnt, cacc.at[pl.ds(s * _SL, _SL)])
            pltpu.sync_copy(ones_h, ones_v)
        plsc.subcore_barrier()
        ebase = s * _EPT

        @pl.loop(0, _NCH)
        def _chunk(j):
            off = ebase + j * _CH
            pltpu.sync_copy(src.at[c, pl.ds(off, _CH)], idxs)
            pltpu.sync_copy(dst.at[c, pl.ds(off, _CH)], idxd)
            pltpu.async_copy(tab.at[idxs], rows, sem).wait()
            pltpu.sync_copy(rows, acc.at[idxd], add=True)
            if with_counts:
                pltpu.sync_copy(ones_v, cacc.at[idxd], add=True)

        plsc.subcore_barrier()
        pltpu.sync_copy(acc.at[pl.ds(s * _SL, _SL), :],
                        out.at[c, pl.ds(s * _SL, _SL), :])
        if with_counts:
            pltpu.sync_copy(cacc.at[pl.ds(s * _SL, _SL)],
                            cnt.at[c, pl.ds(s * _SL, _SL)])

    return segsum


def _make_decoder():
    mesh = plsc.VectorSubcoreMesh(core_axis_name="c", subcore_axis_name="s",
                                  num_cores=_NC, num_subcores=_NS)

    @functools.partial(
        pl.kernel,
        out_type=jax.ShapeDtypeStruct((_NLP,), jnp.float32),
        mesh=mesh,
        scratch_types=[
            pltpu.VMEM((_CH,), jnp.int32),
            pltpu.VMEM((_CH,), jnp.int32),
            pltpu.VMEM((_CH, _D), jnp.float32),
            pltpu.VMEM((_CH, _D), jnp.float32),
            pltpu.VMEM((_CH,), jnp.float32),
            pltpu.VMEM((_D,), jnp.float32),
            pltpu.VMEM((_D,), jnp.float32),
            pltpu.VMEM((_L,), jnp.float32),
            pltpu.SemaphoreType.DMA,
            pltpu.SemaphoreType.DMA,
        ],
    )
    def dec(q, ridx, cidx, w2h, b1h, ph, out,
            iu, ib, bufu, bufb, ob, w2v, b1v, pv, sem1, sem2):
        c = lax.axis_index("c")
        s = lax.axis_index("s")
        pltpu.sync_copy(w2h, w2v)
        pltpu.sync_copy(b1h, b1v)
        pltpu.sync_copy(ph, pv)
        w2r = [w2v[pl.ds(k * _L, _L)] for k in range(_D // _L)]
        b1r = [b1v[pl.ds(k * _L, _L)] for k in range(_D // _L)]
        db2 = pv[0]
        base = (c * _NS + s) * _TPT

        @pl.loop(0, _DCH)
        def _chunk(j):
            off = base + j * _CH
            pltpu.sync_copy(ridx.at[pl.ds(off, _CH)], iu)
            pltpu.sync_copy(cidx.at[pl.ds(off, _CH)], ib)
            du = pltpu.async_copy(q.at[iu], bufu, sem1)
            dv = pltpu.async_copy(q.at[ib], bufb, sem2)
            du.wait()
            dv.wait()

            @pl.loop(0, _CH)
            def _row(r):
                acc = jnp.zeros((_L,), jnp.float32)
                for k in range(_D // _L):
                    v = (bufu[r, pl.ds(k * _L, _L)]
                         + bufb[r, pl.ds(k * _L, _L)] + b1r[k])
                    acc = acc + jnp.maximum(v, 0.0) * w2r[k]
                ob[r] = jnp.sum(acc, axis=0) + db2

            pltpu.sync_copy(ob, out.at[pl.ds(off, _CH)])

    return dec


_segsum_counts = _make_segsum(True)
_segsum_plain = _make_segsum(False)
_decoder = _make_decoder()


def _pad_idx(a, n, fill):
    return jnp.concatenate([a, jnp.full((n - a.shape[0],), fill, jnp.int32)])


def kernel(x_user, x_book, edge_u2b, edge_b2u, edge_label_index, Wu, bu, Wb,
           bb, Wl1_u2b, bl1_u2b, Wr1_u2b, Wl1_b2u, bl1_b2u, Wr1_b2u, Wl2_u2b,
           bl2_u2b, Wr2_u2b, Wl2_b2u, bl2_b2u, Wr2_b2u, dW1, db1, dW2, db2):
    f32 = jnp.float32
    src_ub = _pad_idx(edge_u2b[0].astype(jnp.int32), _EPAD, 0)
    dst_ub = _pad_idx(edge_u2b[1].astype(jnp.int32), _EPAD, _NB)
    src_bu = _pad_idx(edge_b2u[0].astype(jnp.int32) + _NU, _EPAD, _NU)
    dst_bu = _pad_idx(edge_b2u[1].astype(jnp.int32), _EPAD, _NU)
    src_all = jnp.stack([src_ub, src_bu])
    dst_all = jnp.stack([dst_ub, dst_bu])

    zrow = jnp.zeros((_SL, _D), f32)
    zcnt = jnp.zeros((_SL,), f32)
    ones_h = jnp.ones((_CH,), f32)

    a_u, a_b, s_u, s_b = _tc1(x_user, x_book, Wu, bu, Wb, bb,
                              Wl1_u2b, bl1_u2b, Wr1_u2b,
                              Wl1_b2u, bl1_b2u, Wr1_b2u)
    T1 = jnp.concatenate([a_u, a_b], axis=0)

    S1, C1 = _segsum_counts(T1, src_all, dst_all, zrow, zcnt, ones_h)
    cb = C1[0, :_NB].reshape(_NB, 1)
    cu = C1[1, :_NU].reshape(_NU, 1)

    a_u2, a_b2, s_b2, s_u2 = _tc2(S1[0, :_NB], S1[1, :_NU], cb, cu,
                                  s_b, s_u, Wl2_u2b, bl2_u2b, Wr2_u2b,
                                  Wl2_b2u, bl2_b2u, Wr2_b2u)
    T2 = jnp.concatenate([a_u2, a_b2], axis=0)

    (S2,) = _segsum_plain(T2, src_all, dst_all, zrow, zcnt, ones_h)

    qu, qb = _tc3(S2[0, :_NB], S2[1, :_NU], cb, cu, s_b2, s_u2, dW1)
    Q = jnp.concatenate([qu, qb], axis=0)

    ridx = _pad_idx(edge_label_index[0].astype(jnp.int32), _NLP, 0)
    cidx = _pad_idx(edge_label_index[1].astype(jnp.int32) + _NU, _NLP, _NU)
    pars = jnp.zeros((_L,), f32).at[0].set(db2[0])
    res = _decoder(Q, ridx, cidx, dW2.reshape(_D), db1, pars)
    return res[:_NL]


# speedup vs baseline: 4.5537x; 4.5537x over previous
import functools
import jax
import jax.numpy as jnp
from jax import lax
from jax.experimental import pallas as pl
from jax.experimental.pallas import tpu as pltpu
from jax.experimental.pallas import tpu_sc as plsc

_NU = 25000
_NB = 25000
_E = 800000
_NL = 100000
_D = 64
_NC = 2
_NS = 16
_L = 16

_BINS = 25088
_SL = _BINS // _NS
_CH = 128
_NCH = 391
_EPT = _NCH * _CH
_EPAD = _EPT * _NS

_ZB = 224
_TPT = 3200
_DCH = _TPT // _CH
_NLP = _TPT * _NS * _NC

_HI = jax.lax.Precision.HIGHEST


def _dot(a, b):
    return jax.lax.dot_general(a, b, (((1,), (0,)), ((), ())),
                               precision=_HI, preferred_element_type=jnp.float32)


def _tc1_body(xu, xb, Wu, bu, Wb, bb, Wlub, blub, Wrub, Wlbu, blbu, Wrbu,
              au, ab, su, sb):
    hu = _dot(xu[...], Wu[...]) + bu[...]
    hb = _dot(xb[...], Wb[...]) + bb[...]
    au[...] = _dot(hu, Wlub[...])
    ab[...] = _dot(hb, Wlbu[...])
    su[...] = _dot(hu, Wrbu[...]) + blbu[...]
    sb[...] = _dot(hb, Wrub[...]) + blub[...]


def _tc1(xu, xb, Wu, bu, Wb, bb, Wlub, blub, Wrub, Wlbu, blbu, Wrbu):
    n = _NU
    bn = 1000
    grid = (n // bn,)
    row = lambda i: (i, 0)
    full = lambda i: (0, 0)
    out = [jax.ShapeDtypeStruct((n, _D), jnp.float32)] * 4
    return pl.pallas_call(
        _tc1_body,
        grid=grid,
        in_specs=[
            pl.BlockSpec((bn, 3), row),
            pl.BlockSpec((bn, 384), row),
            pl.BlockSpec((3, _D), full),
            pl.BlockSpec((1, _D), full),
            pl.BlockSpec((384, _D), full),
            pl.BlockSpec((1, _D), full),
            pl.BlockSpec((_D, _D), full),
            pl.BlockSpec((1, _D), full),
            pl.BlockSpec((_D, _D), full),
            pl.BlockSpec((_D, _D), full),
            pl.BlockSpec((1, _D), full),
            pl.BlockSpec((_D, _D), full),
        ],
        out_specs=[pl.BlockSpec((bn, _D), row)] * 4,
        out_shape=out,
    )(xu, xb, Wu, bu.reshape(1, _D), Wb, bb.reshape(1, _D),
      Wlub, blub.reshape(1, _D), Wrub, Wlbu, blbu.reshape(1, _D), Wrbu)


def _tc2_body(Sb, Su, cb, cu, sb, su, Wlub, blub, Wrub, Wlbu, blbu, Wrbu,
              au2, ab2, sb2, su2):
    hb = jnp.maximum(Sb[...] / jnp.maximum(cb[...], 1.0) + sb[...], 0.0)
    hu = jnp.maximum(Su[...] / jnp.maximum(cu[...], 1.0) + su[...], 0.0)
    au2[...] = _dot(hu, Wlub[...])
    ab2[...] = _dot(hb, Wlbu[...])
    sb2[...] = _dot(hb, Wrub[...]) + blub[...]
    su2[...] = _dot(hu, Wrbu[...]) + blbu[...]


def _tc2(Sb, Su, cb, cu, sb, su, Wlub, blub, Wrub, Wlbu, blbu, Wrbu):
    n = _NU
    bn = 1000
    grid = (n // bn,)
    row = lambda i: (i, 0)
    full = lambda i: (0, 0)
    out = [jax.ShapeDtypeStruct((n, _D), jnp.float32)] * 4
    return pl.pallas_call(
        _tc2_body,
        grid=grid,
        in_specs=[
            pl.BlockSpec((bn, _D), row),
            pl.BlockSpec((bn, _D), row),
            pl.BlockSpec((bn, 1), row),
            pl.BlockSpec((bn, 1), row),
            pl.BlockSpec((bn, _D), row),
            pl.BlockSpec((bn, _D), row),
            pl.BlockSpec((_D, _D), full),
            pl.BlockSpec((1, _D), full),
            pl.BlockSpec((_D, _D), full),
            pl.BlockSpec((_D, _D), full),
            pl.BlockSpec((1, _D), full),
            pl.BlockSpec((_D, _D), full),
        ],
        out_specs=[pl.BlockSpec((bn, _D), row)] * 4,
        out_shape=out,
    )(Sb, Su, cb, cu, sb, su,
      Wlub, blub.reshape(1, _D), Wrub, Wlbu, blbu.reshape(1, _D), Wrbu)


def _tc3_body(Sb, Su, cb, cu, sb, su, dW1, b1, qu, qb):
    zb = Sb[...] / jnp.maximum(cb[...], 1.0) + sb[...]
    zu = Su[...] / jnp.maximum(cu[...], 1.0) + su[...]
    w = dW1[...]
    qu[...] = _dot(zu, w[:_D, :])
    qb[...] = _dot(zb, w[_D:, :]) + b1[...]


def _tc3(Sb, Su, cb, cu, sb, su, dW1, db1):
    n = _NU
    bn = 1000
    grid = (n // bn,)
    row = lambda i: (i, 0)
    full = lambda i: (0, 0)
    out = [jax.ShapeDtypeStruct((n, _D), jnp.float32)] * 2
    return pl.pallas_call(
        _tc3_body,
        grid=grid,
        in_specs=[
            pl.BlockSpec((bn, _D), row),
            pl.BlockSpec((bn, _D), row),
            pl.BlockSpec((bn, 1), row),
            pl.BlockSpec((bn, 1), row),
            pl.BlockSpec((bn, _D), row),
            pl.BlockSpec((bn, _D), row),
            pl.BlockSpec((2 * _D, _D), full),
            pl.BlockSpec((1, _D), full),
        ],
        out_specs=[pl.BlockSpec((bn, _D), row)] * 2,
        out_shape=out,
    )(Sb, Su, cb, cu, sb, su, dW1, db1.reshape(1, _D))


def _make_segsum(with_counts):
    mesh = plsc.VectorSubcoreMesh(core_axis_name="c", subcore_axis_name="s",
                                  num_cores=_NC, num_subcores=_NS)
    out_type = [jax.ShapeDtypeStruct((_NC, _BINS, _D), jnp.float32)]
    if with_counts:
        out_type.append(jax.ShapeDtypeStruct((_NC * _BINS,), jnp.float32))

    @functools.partial(
        pl.kernel,
        out_type=tuple(out_type),
        mesh=mesh,
        compiler_params=pltpu.CompilerParams(use_tc_tiling_on_sc=False),
        scratch_types=[
            pltpu.VMEM_SHARED((_BINS, _D), jnp.float32),
            pltpu.VMEM_SHARED((_BINS,), jnp.float32),
            pltpu.VMEM((_CH,), jnp.int32),
            pltpu.VMEM((_CH,), jnp.int32),
            pltpu.VMEM((_CH, _D), jnp.float32),
            pltpu.VMEM((_CH,), jnp.float32),
            pltpu.VMEM((_ZB, _D), jnp.float32),
            pltpu.VMEM((_ZB,), jnp.float32),
            pltpu.SemaphoreType.DMA,
        ],
    )
    def segsum(tab, src, dst, zrow, zcnt, ones_h, *rest):
        if with_counts:
            out, cnt, acc, cacc, idxs, idxd, rows, ones_v, zbuf, zcbuf, sem = rest
        else:
            out, acc, cacc, idxs, idxd, rows, ones_v, zbuf, zcbuf, sem = rest
            cnt = None
        c = lax.axis_index("c")
        s = lax.axis_index("s")
        pltpu.sync_copy(zrow, zbuf)
        if with_counts:
            pltpu.sync_copy(zcnt, zcbuf)
            pltpu.sync_copy(ones_h, ones_v)
        for t in range(_SL // _ZB):
            pltpu.sync_copy(zbuf, acc.at[pl.ds(s * _SL + t * _ZB, _ZB), :])
            if with_counts:
                pltpu.sync_copy(zcbuf, cacc.at[pl.ds(s * _SL + t * _ZB, _ZB)])
        plsc.subcore_barrier()
        ebase = c * _EPAD + s * _EPT

        @pl.loop(0, _NCH)
        def _chunk(j):
            off = ebase + j * _CH
            pltpu.sync_copy(src.at[pl.ds(off, _CH)], idxs)
            pltpu.sync_copy(dst.at[pl.ds(off, _CH)], idxd)
            pltpu.async_copy(tab.at[idxs], rows, sem).wait()
            pltpu.sync_copy(rows, acc.at[idxd], add=True)
            if with_counts:
                pltpu.sync_copy(ones_v, cacc.at[idxd], add=True)

        plsc.subcore_barrier()
        for t in range(_SL // _ZB):
            r0 = s * _SL + t * _ZB
            pltpu.sync_copy(acc.at[pl.ds(r0, _ZB), :], zbuf)
            pltpu.sync_copy(zbuf, out.at[c, pl.ds(r0, _ZB), :])
            if with_counts:
                pltpu.sync_copy(cacc.at[pl.ds(r0, _ZB)], zcbuf)
                pltpu.sync_copy(zcbuf, cnt.at[pl.ds(c * _BINS + r0, _ZB)])

    return segsum


def _make_decoder():
    mesh = plsc.VectorSubcoreMesh(core_axis_name="c", subcore_axis_name="s",
                                  num_cores=_NC, num_subcores=_NS)

    @functools.partial(
        pl.kernel,
        out_type=(jax.ShapeDtypeStruct((_NLP, _D), jnp.float32),
                  jax.ShapeDtypeStruct((_NLP, _D), jnp.float32)),
        mesh=mesh,
        compiler_params=pltpu.CompilerParams(use_tc_tiling_on_sc=False),
        scratch_types=[
            pltpu.VMEM((_CH,), jnp.int32),
            pltpu.VMEM((_CH,), jnp.int32),
            pltpu.VMEM((_CH, _D), jnp.float32),
            pltpu.VMEM((_CH, _D), jnp.float32),
            pltpu.SemaphoreType.DMA,
            pltpu.SemaphoreType.DMA,
        ],
    )
    def dec(q, ridx, cidx, out_u, out_b, iu, ib, bufu, bufb, sem1, sem2):
        c = lax.axis_index("c")
        s = lax.axis_index("s")
        base = (c * _NS + s) * _TPT

        @pl.loop(0, _DCH)
        def _chunk(j):
            off = base + j * _CH
            pltpu.sync_copy(ridx.at[pl.ds(off, _CH)], iu)
            pltpu.sync_copy(cidx.at[pl.ds(off, _CH)], ib)
            du = pltpu.async_copy(q.at[iu], bufu, sem1)
            dv = pltpu.async_copy(q.at[ib], bufb, sem2)
            du.wait()
            dv.wait()
            pltpu.sync_copy(bufu, out_u.at[pl.ds(off, _CH), :])
            pltpu.sync_copy(bufb, out_b.at[pl.ds(off, _CH), :])

    return dec


def _dec_reduce_body(gu, gb, w2, b2, o):
    z = jnp.maximum(gu[...] + gb[...], 0.0)
    o[...] = _dot(z, w2[...]) + b2[...]


def _dec_reduce(gu, gb, dW2, db2):
    bn = 2048
    grid = (_NLP // bn,)
    row = lambda i: (i, 0)
    full = lambda i: (0, 0)
    return pl.pallas_call(
        _dec_reduce_body,
        grid=grid,
        in_specs=[
            pl.BlockSpec((bn, _D), row),
            pl.BlockSpec((bn, _D), row),
            pl.BlockSpec((_D, 1), full),
            pl.BlockSpec((1, 1), full),
        ],
        out_specs=pl.BlockSpec((bn, 1), row),
        out_shape=jax.ShapeDtypeStruct((_NLP, 1), jnp.float32),
    )(gu, gb, dW2, db2.reshape(1, 1))


_segsum_counts = _make_segsum(True)
_segsum_plain = _make_segsum(False)
_decoder = _make_decoder()


def _pad_idx(a, n, fill):
    return jnp.concatenate([a, jnp.full((n - a.shape[0],), fill, jnp.int32)])


def kernel(x_user, x_book, edge_u2b, edge_b2u, edge_label_index, Wu, bu, Wb,
           bb, Wl1_u2b, bl1_u2b, Wr1_u2b, Wl1_b2u, bl1_b2u, Wr1_b2u, Wl2_u2b,
           bl2_u2b, Wr2_u2b, Wl2_b2u, bl2_b2u, Wr2_b2u, dW1, db1, dW2, db2):
    f32 = jnp.float32
    src_ub = _pad_idx(edge_u2b[0].astype(jnp.int32), _EPAD, 0)
    dst_ub = _pad_idx(edge_u2b[1].astype(jnp.int32), _EPAD, _NB)
    src_bu = _pad_idx(edge_b2u[0].astype(jnp.int32) + _NU, _EPAD, _NU)
    dst_bu = _pad_idx(edge_b2u[1].astype(jnp.int32), _EPAD, _NU)
    src_all = jnp.concatenate([src_ub, src_bu])
    dst_all = jnp.concatenate([dst_ub, dst_bu])

    zrow = jnp.zeros((_ZB, _D), f32)
    zcnt = jnp.zeros((_ZB,), f32)
    ones_h = jnp.ones((_CH,), f32)

    a_u, a_b, s_u, s_b = _tc1(x_user, x_book, Wu, bu, Wb, bb,
                              Wl1_u2b, bl1_u2b, Wr1_u2b,
                              Wl1_b2u, bl1_b2u, Wr1_b2u)
    T1 = jnp.concatenate([a_u, a_b], axis=0)

    S1, C1 = _segsum_counts(T1, src_all, dst_all, zrow, zcnt, ones_h)
    cb = C1[:_NB].reshape(_NB, 1)
    cu = C1[_BINS:_BINS + _NU].reshape(_NU, 1)

    a_u2, a_b2, s_b2, s_u2 = _tc2(S1[0, :_NB], S1[1, :_NU], cb, cu,
                                  s_b, s_u, Wl2_u2b, bl2_u2b, Wr2_u2b,
                                  Wl2_b2u, bl2_b2u, Wr2_b2u)
    T2 = jnp.concatenate([a_u2, a_b2], axis=0)

    (S2,) = _segsum_plain(T2, src_all, dst_all, zrow, zcnt, ones_h)

    qu, qb = _tc3(S2[0, :_NB], S2[1, :_NU], cb, cu, s_b2, s_u2, dW1, db1)
    Q = jnp.concatenate([qu, qb], axis=0)

    ridx = _pad_idx(edge_label_index[0].astype(jnp.int32), _NLP, 0)
    cidx = _pad_idx(edge_label_index[1].astype(jnp.int32) + _NU, _NLP, _NU)
    gu, gb = _decoder(Q, ridx, cidx)
    res = _dec_reduce(gu, gb, dW2, db2)
    return res[:_NL, 0]


# speedup vs baseline: 7.3681x; 1.6180x over previous
import functools
import jax
import jax.numpy as jnp
from jax import lax
from jax.experimental import pallas as pl
from jax.experimental.pallas import tpu as pltpu
from jax.experimental.pallas import tpu_sc as plsc

_NU = 25000
_NB = 25000
_E = 800000
_NL = 100000
_D = 64
_NC = 2
_NS = 16
_L = 16

_BINS = 25088
_SL = _BINS // _NS
_CH = 128
_BC = 14
_NBLK = 28
_NCH = _BC * _NBLK
_EPT = _NCH * _CH
_EPAD = _EPT * _NS
_NBUF = 2
_ZB = 112

_DC = 128
_TPT = 3200
_DCH = _TPT // _DC
_NLP = _TPT * _NS * _NC

_HI = jax.lax.Precision.HIGHEST


def _dot(a, b):
    return jax.lax.dot_general(a, b, (((1,), (0,)), ((), ())),
                               precision=_HI, preferred_element_type=jnp.float32)


def _tc1_body(xu, xb, Wu, bu, Wb, bb, Wlub, blub, Wrub, Wlbu, blbu, Wrbu,
              au, ab, su, sb):
    hu = _dot(xu[...], Wu[...]) + bu[...]
    hb = _dot(xb[...], Wb[...]) + bb[...]
    au[...] = _dot(hu, Wlub[...])
    ab[...] = _dot(hb, Wlbu[...])
    su[...] = _dot(hu, Wrbu[...]) + blbu[...]
    sb[...] = _dot(hb, Wrub[...]) + blub[...]


def _tc1(xu, xb, Wu, bu, Wb, bb, Wlub, blub, Wrub, Wlbu, blbu, Wrbu):
    n = _NU
    bn = 1000
    grid = (n // bn,)
    row = lambda i: (i, 0)
    full = lambda i: (0, 0)
    out = [jax.ShapeDtypeStruct((n, _D), jnp.float32)] * 4
    return pl.pallas_call(
        _tc1_body,
        grid=grid,
        in_specs=[
            pl.BlockSpec((bn, 3), row),
            pl.BlockSpec((bn, 384), row),
            pl.BlockSpec((3, _D), full),
            pl.BlockSpec((1, _D), full),
            pl.BlockSpec((384, _D), full),
            pl.BlockSpec((1, _D), full),
            pl.BlockSpec((_D, _D), full),
            pl.BlockSpec((1, _D), full),
            pl.BlockSpec((_D, _D), full),
            pl.BlockSpec((_D, _D), full),
            pl.BlockSpec((1, _D), full),
            pl.BlockSpec((_D, _D), full),
        ],
        out_specs=[pl.BlockSpec((bn, _D), row)] * 4,
        out_shape=out,
    )(xu, xb, Wu, bu.reshape(1, _D), Wb, bb.reshape(1, _D),
      Wlub, blub.reshape(1, _D), Wrub, Wlbu, blbu.reshape(1, _D), Wrbu)


def _tc2_body(Sb, Su, cb, cu, sb, su, Wlub, blub, Wrub, Wlbu, blbu, Wrbu,
              au2, ab2, sb2, su2):
    hb = jnp.maximum(Sb[...] / jnp.maximum(cb[...], 1.0) + sb[...], 0.0)
    hu = jnp.maximum(Su[...] / jnp.maximum(cu[...], 1.0) + su[...], 0.0)
    au2[...] = _dot(hu, Wlub[...])
    ab2[...] = _dot(hb, Wlbu[...])
    sb2[...] = _dot(hb, Wrub[...]) + blub[...]
    su2[...] = _dot(hu, Wrbu[...]) + blbu[...]


def _tc2(Sb, Su, cb, cu, sb, su, Wlub, blub, Wrub, Wlbu, blbu, Wrbu):
    n = _NU
    bn = 1000
    grid = (n // bn,)
    row = lambda i: (i, 0)
    full = lambda i: (0, 0)
    out = [jax.ShapeDtypeStruct((n, _D), jnp.float32)] * 4
    return pl.pallas_call(
        _tc2_body,
        grid=grid,
        in_specs=[
            pl.BlockSpec((bn, _D), row),
            pl.BlockSpec((bn, _D), row),
            pl.BlockSpec((bn, 1), row),
            pl.BlockSpec((bn, 1), row),
            pl.BlockSpec((bn, _D), row),
            pl.BlockSpec((bn, _D), row),
            pl.BlockSpec((_D, _D), full),
            pl.BlockSpec((1, _D), full),
            pl.BlockSpec((_D, _D), full),
            pl.BlockSpec((_D, _D), full),
            pl.BlockSpec((1, _D), full),
            pl.BlockSpec((_D, _D), full),
        ],
        out_specs=[pl.BlockSpec((bn, _D), row)] * 4,
        out_shape=out,
    )(Sb, Su, cb, cu, sb, su,
      Wlub, blub.reshape(1, _D), Wrub, Wlbu, blbu.reshape(1, _D), Wrbu)


def _tc3_body(Sb, Su, cb, cu, sb, su, dW1, b1, qu, qb):
    zb = Sb[...] / jnp.maximum(cb[...], 1.0) + sb[...]
    zu = Su[...] / jnp.maximum(cu[...], 1.0) + su[...]
    w = dW1[...]
    qu[...] = _dot(zu, w[:_D, :])
    qb[...] = _dot(zb, w[_D:, :]) + b1[...]


def _tc3(Sb, Su, cb, cu, sb, su, dW1, db1):
    n = _NU
    bn = 1000
    grid = (n // bn,)
    row = lambda i: (i, 0)
    full = lambda i: (0, 0)
    out = [jax.ShapeDtypeStruct((n, _D), jnp.float32)] * 2
    return pl.pallas_call(
        _tc3_body,
        grid=grid,
        in_specs=[
            pl.BlockSpec((bn, _D), row),
            pl.BlockSpec((bn, _D), row),
            pl.BlockSpec((bn, 1), row),
            pl.BlockSpec((bn, 1), row),
            pl.BlockSpec((bn, _D), row),
            pl.BlockSpec((bn, _D), row),
            pl.BlockSpec((2 * _D, _D), full),
            pl.BlockSpec((1, _D), full),
        ],
        out_specs=[pl.BlockSpec((bn, _D), row)] * 2,
        out_shape=out,
    )(Sb, Su, cb, cu, sb, su, dW1, db1.reshape(1, _D))


def _make_segsum(with_counts):
    mesh = plsc.VectorSubcoreMesh(core_axis_name="c", subcore_axis_name="s",
                                  num_cores=_NC, num_subcores=_NS)
    out_type = [jax.ShapeDtypeStruct((_NC, _BINS, _D), jnp.float32)]
    if with_counts:
        out_type.append(jax.ShapeDtypeStruct((_NC * _BINS,), jnp.float32))

    @functools.partial(
        pl.kernel,
        out_type=tuple(out_type),
        mesh=mesh,
        compiler_params=pltpu.CompilerParams(use_tc_tiling_on_sc=False),
        scratch_types=[
            pltpu.VMEM_SHARED((_BINS, _D), jnp.float32),
            pltpu.VMEM_SHARED((_BINS,), jnp.float32),
            [pltpu.VMEM((_BC, _CH), jnp.int32)] * 2,
            [pltpu.VMEM((_BC, _CH), jnp.int32)] * 2,
            [pltpu.VMEM((_CH, _D), jnp.float32)] * _NBUF,
            pltpu.VMEM((_CH,), jnp.float32),
            pltpu.VMEM((_ZB,), jnp.float32),
            [pltpu.SemaphoreType.DMA] * 2,
            [pltpu.SemaphoreType.DMA] * _NBUF,
        ],
    )
    def segsum(tab, src2, dst2, zrow, zcnt, ones_h, *rest):
        if with_counts:
            out, cnt, acc, cacc, idxs, idxd, rows, ones_v, czbuf, isem, gsem = rest
        else:
            out, acc, cacc, idxs, idxd, rows, ones_v, czbuf, isem, gsem = rest
            cnt = None
        c = lax.axis_index("c")
        s = lax.axis_index("s")
        zv = rows[0].at[pl.ds(0, _ZB), :]
        pltpu.sync_copy(zrow, zv)
        if with_counts:
            pltpu.sync_copy(zcnt, czbuf)
            pltpu.sync_copy(ones_h, ones_v)
        zd = []
        for t in range(_SL // _ZB):
            r0 = s * _SL + t * _ZB
            zd.append(pltpu.async_copy(zv, acc.at[pl.ds(r0, _ZB), :], gsem[0]))
            if with_counts:
                zd.append(pltpu.async_copy(czbuf, cacc.at[pl.ds(r0, _ZB)],
                                           gsem[1]))
        for d in zd:
            d.wait()
        plsc.subcore_barrier()

        row0 = c * (_EPAD // _CH) + s * (_EPT // _CH)

        def scatter(p, k, b):
            pltpu.make_async_copy(tab.at[pl.ds(0, _CH), :], rows[b],
                                  gsem[b]).wait()
            pltpu.sync_copy(rows[b], acc.at[idxd[p].at[k]], add=True)
            if with_counts:
                pltpu.sync_copy(ones_v, cacc.at[idxd[p].at[k]], add=True)

        def process_block(p):
            for b in range(_NBUF):
                pltpu.async_copy(tab.at[idxs[p].at[b]], rows[b], gsem[b])

            @pl.loop(0, _BC - _NBUF, step=_NBUF)
            def _grp(k0):
                for b in range(_NBUF):
                    scatter(p, k0 + b, b)
                    pltpu.async_copy(tab.at[idxs[p].at[k0 + b + _NBUF]],
                                     rows[b], gsem[b])

            for b in range(_NBUF):
                scatter(p, _BC - _NBUF + b, b)

        def wait_idx(p):
            pltpu.make_async_copy(src2.at[pl.ds(0, _BC), :], idxs[p],
                                  isem[p]).wait()
            pltpu.make_async_copy(dst2.at[pl.ds(0, _BC), :], idxd[p],
                                  isem[p]).wait()

        def issue_idx(bn, p):
            rn = row0 + bn * _BC
            pltpu.async_copy(src2.at[pl.ds(rn, _BC), :], idxs[p], isem[p])
            pltpu.async_copy(dst2.at[pl.ds(rn, _BC), :], idxd[p], isem[p])

        issue_idx(0, 0)
        issue_idx(1, 1)

        @pl.loop(0, _NBLK, step=2)
        def _blk(B):
            for p in range(2):
                wait_idx(p)
                process_block(p)

                @pl.when(B + p + 2 < _NBLK)
                def _():
                    issue_idx(B + p + 2, p)

        plsc.subcore_barrier()
        wd = [None, None]
        for t in range(_SL // _ZB):
            b = t % 2
            r0 = s * _SL + t * _ZB
            bv = rows[b].at[pl.ds(0, _ZB), :]
            if wd[b] is not None:
                wd[b].wait()
            pltpu.sync_copy(acc.at[pl.ds(r0, _ZB), :], bv)
            wd[b] = pltpu.async_copy(bv, out.at[c, pl.ds(r0, _ZB), :], gsem[b])
            if with_counts:
                pltpu.sync_copy(cacc.at[pl.ds(r0, _ZB)], czbuf)
                pltpu.sync_copy(czbuf, cnt.at[pl.ds(c * _BINS + r0, _ZB)])
        for d in wd:
            if d is not None:
                d.wait()

    return segsum


def _make_decoder():
    mesh = plsc.VectorSubcoreMesh(core_axis_name="c", subcore_axis_name="s",
                                  num_cores=_NC, num_subcores=_NS)

    @functools.partial(
        pl.kernel,
        out_type=(jax.ShapeDtypeStruct((_NLP, _D), jnp.float32),
                  jax.ShapeDtypeStruct((_NLP, _D), jnp.float32)),
        mesh=mesh,
        compiler_params=pltpu.CompilerParams(use_tc_tiling_on_sc=False),
        scratch_types=[
            pltpu.VMEM((_DC,), jnp.int32),
            pltpu.VMEM((_DC,), jnp.int32),
            pltpu.VMEM((_DC, _D), jnp.float32),
            pltpu.VMEM((_DC, _D), jnp.float32),
            pltpu.SemaphoreType.DMA,
            pltpu.SemaphoreType.DMA,
        ],
    )
    def dec(q, ridx, cidx, out_u, out_b, iu, ib, bufu, bufb, sem1, sem2):
        c = lax.axis_index("c")
        s = lax.axis_index("s")
        base = (c * _NS + s) * _TPT

        @pl.loop(0, _DCH)
        def _chunk(j):
            off = base + j * _DC
            pltpu.sync_copy(ridx.at[pl.ds(off, _DC)], iu)
            pltpu.sync_copy(cidx.at[pl.ds(off, _DC)], ib)
            du = pltpu.async_copy(q.at[iu], bufu, sem1)
            dv = pltpu.async_copy(q.at[ib], bufb, sem2)
            du.wait()
            dv.wait()
            pltpu.sync_copy(bufu, out_u.at[pl.ds(off, _DC), :])
            pltpu.sync_copy(bufb, out_b.at[pl.ds(off, _DC), :])

    return dec


def _dec_reduce_body(gu, gb, w2, b2, o):
    z = jnp.maximum(gu[...] + gb[...], 0.0)
    o[...] = _dot(z, w2[...]) + b2[...]


def _dec_reduce(gu, gb, dW2, db2):
    bn = 2048
    grid = (_NLP // bn,)
    row = lambda i: (i, 0)
    full = lambda i: (0, 0)
    return pl.pallas_call(
        _dec_reduce_body,
        grid=grid,
        in_specs=[
            pl.BlockSpec((bn, _D), row),
            pl.BlockSpec((bn, _D), row),
            pl.BlockSpec((_D, 1), full),
            pl.BlockSpec((1, 1), full),
        ],
        out_specs=pl.BlockSpec((bn, 1), row),
        out_shape=jax.ShapeDtypeStruct((_NLP, 1), jnp.float32),
    )(gu, gb, dW2, db2.reshape(1, 1))


_segsum_counts = _make_segsum(True)
_segsum_plain = _make_segsum(False)
_decoder = _make_decoder()


def _pad_idx(a, n, fill):
    return jnp.concatenate([a, jnp.full((n - a.shape[0],), fill, jnp.int32)])


def kernel(x_user, x_book, edge_u2b, edge_b2u, edge_label_index, Wu, bu, Wb,
           bb, Wl1_u2b, bl1_u2b, Wr1_u2b, Wl1_b2u, bl1_b2u, Wr1_b2u, Wl2_u2b,
           bl2_u2b, Wr2_u2b, Wl2_b2u, bl2_b2u, Wr2_b2u, dW1, db1, dW2, db2):
    f32 = jnp.float32
    src_ub = _pad_idx(edge_u2b[0].astype(jnp.int32), _EPAD, 0)
    dst_ub = _pad_idx(edge_u2b[1].astype(jnp.int32), _EPAD, _NB)
    src_bu = _pad_idx(edge_b2u[0].astype(jnp.int32) + _NU, _EPAD, _NU)
    dst_bu = _pad_idx(edge_b2u[1].astype(jnp.int32), _EPAD, _NU)
    src_all = jnp.concatenate([src_ub, src_bu]).reshape(-1, _CH)
    dst_all = jnp.concatenate([dst_ub, dst_bu]).reshape(-1, _CH)

    zrow = jnp.zeros((_ZB, _D), f32)
    zcnt = jnp.zeros((_ZB,), f32)
    ones_h = jnp.ones((_CH,), f32)

    a_u, a_b, s_u, s_b = _tc1(x_user, x_book, Wu, bu, Wb, bb,
                              Wl1_u2b, bl1_u2b, Wr1_u2b,
                              Wl1_b2u, bl1_b2u, Wr1_b2u)
    T1 = jnp.concatenate([a_u, a_b], axis=0)

    S1, C1 = _segsum_counts(T1, src_all, dst_all, zrow, zcnt, ones_h)
    cb = C1[:_NB].reshape(_NB, 1)
    cu = C1[_BINS:_BINS + _NU].reshape(_NU, 1)

    a_u2, a_b2, s_b2, s_u2 = _tc2(S1[0, :_NB], S1[1, :_NU], cb, cu,
                                  s_b, s_u, Wl2_u2b, bl2_u2b, Wr2_u2b,
                                  Wl2_b2u, bl2_b2u, Wr2_b2u)
    T2 = jnp.concatenate([a_u2, a_b2], axis=0)

    (S2,) = _segsum_plain(T2, src_all, dst_all, zrow, zcnt, ones_h)

    qu, qb = _tc3(S2[0, :_NB], S2[1, :_NU], cb, cu, s_b2, s_u2, dW1, db1)
    Q = jnp.concatenate([qu, qb], axis=0)

    ridx = _pad_idx(edge_label_index[0].astype(jnp.int32), _NLP, 0)
    cidx = _pad_idx(edge_label_index[1].astype(jnp.int32) + _NU, _NLP, _NU)
    gu, gb = _decoder(Q, ridx, cidx)
    res = _dec_reduce(gu, gb, dW2, db2)
    return res[:_NL, 0]
